# baseline (speedup 1.0000x reference)
import functools

import jax
import jax.numpy as jnp
from jax import lax
from jax.experimental import pallas as pl
from jax.experimental.pallas import tpu as pltpu
from jax.experimental.pallas import tpu_sc as plsc

_ACTION = 1000
_N = 1_000_000
_NC = 1
_NW = 16 * _NC
_PER_W = 999_936 // _NW
_TAIL = _N - _NW * _PER_W
_CHUNK = 3_472
_NCH = _PER_W // _CHUNK


@functools.partial(
    pl.kernel,
    mesh=plsc.VectorSubcoreMesh(
        core_axis_name="c", subcore_axis_name="s", num_cores=_NC
    ),
    out_type=jax.ShapeDtypeStruct((_N,), jnp.float32),
    scratch_types=[
        pltpu.VMEM((_CHUNK,), jnp.float32),
        pltpu.VMEM((16,), jnp.int32),
        pltpu.VMEM((16,), jnp.float32),
        pltpu.SemaphoreType.DMA,
        pltpu.SemaphoreType.DMA,
        pltpu.SemaphoreType.DMA,
    ],
)
def _one_hot_sc(inp_hbm, out_hbm, zbuf, ivmem, onebuf, sem, sem_first, sem_inp):
    wid = lax.axis_index("s") * _NC + lax.axis_index("c")
    base = wid * _PER_W

    inp_copy = pltpu.async_copy(inp_hbm, ivmem.at[pl.ds(0, 2)], sem_inp)

    zeros16 = jnp.zeros((16,), jnp.float32)
    for j in range(_CHUNK // 16):
        zbuf[pl.ds(j * 16, 16)] = zeros16

    inp_copy.wait()
    iv = ivmem[...]
    index = iv[0] * _ACTION + iv[1]

    q = index // _PER_W
    owner = jnp.where(q >= _NW, 0, q)
    rel = index - owner * _PER_W
    k_own = jnp.where(rel < _PER_W, rel, 0) // _CHUNK

    first = pltpu.async_copy(
        zbuf, out_hbm.at[pl.ds(base + k_own * _CHUNK, _CHUNK)], sem_first
    )
    rest = [
        pltpu.async_copy(
            zbuf,
            out_hbm.at[
                pl.ds(base + ((k_own + 1 + i) % _NCH) * _CHUNK, _CHUNK)
            ],
            sem,
        )
        for i in range(_NCH - 1)
    ]

    @pl.when(wid == 0)
    def _():
        pltpu.async_copy(
            zbuf.at[pl.ds(0, _TAIL)], out_hbm.at[pl.ds(_NW * _PER_W, _TAIL)], sem
        ).wait()

    @pl.when(wid == owner)
    def _():
        first.wait()
        base16 = (index // 16) * 16
        lane = index - base16
        onebuf[...] = jnp.where(
            lax.iota(jnp.int32, 16) == lane, 1.0, 0.0
        ).astype(jnp.float32)
        pltpu.sync_copy(onebuf, out_hbm.at[pl.ds(base16, 16)])

    @pl.when(wid != owner)
    def _():
        first.wait()

    for c in rest:
        c.wait()


def kernel(inp):
    return _one_hot_sc(inp)


# speedup vs baseline: 1.0372x; 1.0372x over previous
import functools

import jax
import jax.numpy as jnp
from jax import lax
from jax.experimental import pallas as pl
from jax.experimental.pallas import tpu as pltpu
from jax.experimental.pallas import tpu_sc as plsc

_ACTION = 1000
_N = 1_000_000
_NC = 1
_NW = 16 * _NC
_PER_W = 999_936 // _NW
_TAIL = _N - _NW * _PER_W
_CHUNK = 3_472
_NCH = _PER_W // _CHUNK


@functools.partial(
    pl.kernel,
    mesh=plsc.VectorSubcoreMesh(
        core_axis_name="c", subcore_axis_name="s", num_cores=_NC
    ),
    out_type=jax.ShapeDtypeStruct((_N,), jnp.float32),
    scratch_types=[
        pltpu.VMEM((_CHUNK,), jnp.float32),
        pltpu.VMEM((16,), jnp.int32),
        pltpu.VMEM((16,), jnp.float32),
        pltpu.SemaphoreType.DMA,
        pltpu.SemaphoreType.DMA,
    ],
)
def _one_hot_sc(inp_hbm, out_hbm, zbuf, ivmem, onebuf, sem, sem_inp):
    wid = lax.axis_index("s") * _NC + lax.axis_index("c")
    base = wid * _PER_W

    inp_copy = pltpu.async_copy(inp_hbm, ivmem.at[pl.ds(0, 2)], sem_inp)

    zeros16 = jnp.zeros((16,), jnp.float32)
    for j in range(_CHUNK // 16):
        zbuf[pl.ds(j * 16, 16)] = zeros16

    copies = [
        pltpu.async_copy(zbuf, out_hbm.at[pl.ds(base + i * _CHUNK, _CHUNK)], sem)
        for i in range(_NCH)
    ]

    @pl.when(wid == 0)
    def _():
        copies.append(
            pltpu.async_copy(
                zbuf.at[pl.ds(0, _TAIL)], out_hbm.at[pl.ds(_NW * _PER_W, _TAIL)], sem
            )
        )

    inp_copy.wait()
    iv = ivmem[...]
    index = iv[0] * _ACTION + iv[1]
    q = index // _PER_W
    owner = jnp.where(q >= _NW, 0, q)
    base16 = (index // 16) * 16
    lane = index - base16
    onebuf[...] = jnp.where(lax.iota(jnp.int32, 16) == lane, 1.0, 0.0).astype(
        jnp.float32
    )

    for c in copies[:_NCH]:
        c.wait()

    @pl.when(wid == 0)
    def _():
        pltpu.make_async_copy(
            zbuf.at[pl.ds(0, _TAIL)], out_hbm.at[pl.ds(_NW * _PER_W, _TAIL)], sem
        ).wait()

    @pl.when(wid == owner)
    def _():
        pltpu.sync_copy(onebuf, out_hbm.at[pl.ds(base16, 16)])


def kernel(inp):
    return _one_hot_sc(inp)
